# baseline (speedup 1.0000x reference)
import functools

import jax
import jax.numpy as jnp
import numpy as np
from jax import lax
from jax.experimental import pallas as pl
from jax.experimental.pallas import tpu as pltpu
from jax.experimental.pallas import tpu_sc as plsc

B = 4096
F = 26
V = 100000
D = 64
DENSE = 13
H1 = 1024
H2 = 512

_N = B * F
_NW = 16
_PER_W = _N // _NW
_CHUNK = 128
_NCHUNK = _PER_W // _CHUNK


def _gather_rows(flat_tables, flat_idx):
    mesh = plsc.VectorSubcoreMesh(core_axis_name="c", subcore_axis_name="s", num_cores=1)

    @functools.partial(
        pl.kernel,
        out_type=jax.ShapeDtypeStruct((_N, D), jnp.float32),
        mesh=mesh,
        scratch_types=[
            pltpu.VMEM((_PER_W,), jnp.int32),
            pltpu.VMEM((2, _CHUNK, D), jnp.float32),
            pltpu.SemaphoreType.DMA((2,)),
        ],
        compiler_params=pltpu.CompilerParams(use_tc_tiling_on_sc=False),
    )
    def gather_kernel(tab_hbm, idx_hbm, out_hbm, idx_v, buf, sem):
        wid = lax.axis_index("s")
        base = wid * _PER_W
        pltpu.sync_copy(idx_hbm.at[pl.ds(base, _PER_W)], idx_v)

        pltpu.async_copy(
            tab_hbm.at[idx_v.at[pl.ds(0, _CHUNK)]], buf.at[0], sem.at[0]
        )

        def body(c, carry):
            slot = lax.rem(c, 2)
            nxt = lax.rem(c + 1, 2)

            @pl.when(c + 1 < _NCHUNK)
            def _():
                pltpu.async_copy(
                    tab_hbm.at[idx_v.at[pl.ds((c + 1) * _CHUNK, _CHUNK)]],
                    buf.at[nxt], sem.at[nxt],
                )

            pltpu.make_async_copy(
                tab_hbm.at[pl.ds(0, _CHUNK)], buf.at[slot], sem.at[slot]
            ).wait()
            pltpu.sync_copy(
                buf.at[slot], out_hbm.at[pl.ds(base + c * _CHUNK, _CHUNK)]
            )
            return carry

        lax.fori_loop(0, _NCHUNK, body, 0)

    return gather_kernel(flat_tables, flat_idx)


_BB = 512


def _mlp_body(x_ref, dense_ref, w1_ref, w1d_ref, b1_ref, w2_ref, b2_ref,
              w3_ref, b3_ref, out_ref):
    h = jnp.dot(x_ref[...].astype(jnp.bfloat16), w1_ref[...],
                preferred_element_type=jnp.float32)
    h = h + jnp.dot(dense_ref[...], w1d_ref[...],
                    preferred_element_type=jnp.float32)
    h = jnp.maximum(h + b1_ref[...], 0.0).astype(jnp.bfloat16)
    h2 = jnp.dot(h, w2_ref[...], preferred_element_type=jnp.float32)
    h2 = jnp.maximum(h2 + b2_ref[...], 0.0)
    logit = jnp.sum(h2 * w3_ref[...], axis=1, keepdims=True) + b3_ref[...]
    out_ref[...] = jax.nn.sigmoid(logit)


def _mlp(x, dense, W1a, W1d, b1, W2, b2, w3row, b3):
    return pl.pallas_call(
        _mlp_body,
        grid=(B // _BB,),
        in_specs=[
            pl.BlockSpec((_BB, F * D), lambda i: (i, 0)),
            pl.BlockSpec((_BB, DENSE), lambda i: (i, 0)),
            pl.BlockSpec((F * D, H1), lambda i: (0, 0)),
            pl.BlockSpec((DENSE, H1), lambda i: (0, 0)),
            pl.BlockSpec((1, H1), lambda i: (0, 0)),
            pl.BlockSpec((H1, H2), lambda i: (0, 0)),
            pl.BlockSpec((1, H2), lambda i: (0, 0)),
            pl.BlockSpec((1, H2), lambda i: (0, 0)),
            pl.BlockSpec((1, 1), lambda i: (0, 0)),
        ],
        out_specs=pl.BlockSpec((_BB, 1), lambda i: (i, 0)),
        out_shape=jax.ShapeDtypeStruct((B, 1), jnp.float32),
    )(x, dense, W1a, W1d, b1, W2, b2, w3row, b3)


def kernel(sparse_ids, dense_feats, tables, W1, b1, W2, b2, W3, b3):
    flat_tables = tables.reshape(F * V, D)
    offs = (jnp.arange(F, dtype=jnp.int32) * V)[None, :]
    flat_idx = (sparse_ids.astype(jnp.int32) + offs).reshape(_N)

    x = _gather_rows(flat_tables, flat_idx).reshape(B, F * D)

    W1a = W1[:F * D].astype(jnp.bfloat16)
    W1d = W1[F * D:]
    W2b = W2.astype(jnp.bfloat16)

    return _mlp(x, dense_feats, W1a, W1d, b1.reshape(1, H1), W2b,
                b2.reshape(1, H2), W3.reshape(1, H2), b3.reshape(1, 1))
